# baseline (speedup 1.0000x reference)
import math
import jax
import jax.numpy as jnp
from jax import lax
from jax.experimental import pallas as pl
from jax.experimental.pallas import tpu as pltpu
from jax.experimental.pallas import tpu_sc as plsc

_S = 64.0
_M = 0.5
_COS_M = math.cos(_M)
_SIN_M = math.sin(_M)
_THRESHOLD = math.cos(math.pi - _M)
_MM = math.sin(math.pi - _M) * _M

_B = 1024
_C = 100000

_TC_ROWS = 0
_SC_ROWS = _B - _TC_ROWS
_RB = 32

_NC = 2
_NS = 16
_NW = _NC * _NS
_GPW = _B // _NW
_SC_RPT = _SC_ROWS // _NW if _SC_ROWS else 0
_SC_CH = _C // 2
_SC_VREGS = _SC_CH // 16


def _sc_gather_body(labels_hbm, logits_flat_hbm, out_hbm, lbl_v, idx_v, val_v, sem):
    wid = lax.axis_index("s") * _NC + lax.axis_index("c")
    base = wid * _GPW
    pltpu.sync_copy(labels_hbm.at[pl.ds(base, _GPW)], lbl_v)
    for c in range(_GPW // 16):
        rows = base + c * 16 + lax.iota(jnp.int32, 16)
        idx_v[pl.ds(c * 16, 16)] = rows * _C + lbl_v[pl.ds(c * 16, 16)]
    pltpu.async_copy(logits_flat_hbm.at[idx_v], val_v, sem).wait()
    pltpu.sync_copy(val_v, out_hbm.at[pl.ds(base, _GPW)])


def _gather_targets(logits, labels):
    run = pl.kernel(
        _sc_gather_body,
        out_type=jax.ShapeDtypeStruct((_B,), jnp.float32),
        mesh=plsc.VectorSubcoreMesh(core_axis_name="c", subcore_axis_name="s"),
        scratch_types=[
            pltpu.VMEM((_GPW,), jnp.int32),
            pltpu.VMEM((_GPW,), jnp.int32),
            pltpu.VMEM((_GPW,), jnp.float32),
            pltpu.SemaphoreType.DMA,
        ],
    )
    return run(labels, logits.reshape(_B * _C))


def _tc_prep_body(target_ref, t_ref, ctm_ref, num_ref, corr_ref, tnew_ref):
    tl = target_ref[:, :]
    t_new = 0.01 * jnp.mean(tl) + 0.99 * t_ref[0]
    sin_theta = jnp.sqrt(1.0 - tl * tl)
    ctm = tl * _COS_M - sin_theta * _SIN_M
    final = jnp.where(tl > _THRESHOLD, ctm, tl - _MM)
    num = _S * final
    mod_tl = jnp.where(tl > ctm, tl * (t_new + tl), tl)
    ctm_ref[:, :] = jnp.broadcast_to(ctm, (_B, 16))
    num_ref[:, :] = num
    corr_ref[:, :] = jnp.exp(num) - jnp.exp(_S * mod_tl)
    tnew_ref[:, :] = t_new * jnp.ones((1, 16), jnp.float32)


def _tc_prep(target2d, t):
    return pl.pallas_call(
        _tc_prep_body,
        in_specs=[
            pl.BlockSpec((_B, 1), lambda: (0, 0)),
            pl.BlockSpec(memory_space=pltpu.SMEM),
        ],
        out_specs=[
            pl.BlockSpec((_B, 16), lambda: (0, 0)),
            pl.BlockSpec((_B, 1), lambda: (0, 0)),
            pl.BlockSpec((_B, 1), lambda: (0, 0)),
            pl.BlockSpec((1, 16), lambda: (0, 0)),
        ],
        out_shape=[
            jax.ShapeDtypeStruct((_B, 16), jnp.float32),
            jax.ShapeDtypeStruct((_B, 1), jnp.float32),
            jax.ShapeDtypeStruct((_B, 1), jnp.float32),
            jax.ShapeDtypeStruct((1, 16), jnp.float32),
        ],
    )(target2d, t)


def _sc_rowsum_body(ctm_hbm, tnew_hbm, logits_flat_hbm, out_hbm,
                    ctm_v, tnew_v, buf0, buf1, out_v, sem0, sem1):
    wid = lax.axis_index("s") * _NC + lax.axis_index("c")
    rbase = _TC_ROWS + wid * _SC_RPT
    pltpu.sync_copy(ctm_hbm.at[pl.ds(rbase * 16, _SC_RPT * 16)], ctm_v)
    pltpu.sync_copy(tnew_hbm, tnew_v)
    tn = tnew_v[...]

    def _chunk_sum(buf, ctm_vec, acc0):
        def inner(j, acc):
            v = buf[pl.ds(j * 16, 16)]
            xm = jnp.where(v > ctm_vec, v * (tn + v), v)
            return acc + jnp.exp(_S * xm)
        return lax.fori_loop(0, _SC_VREGS, inner, acc0)

    def _issue(row, half, buf, sem):
        base = row * _C + half * _SC_CH
        return pltpu.async_copy(logits_flat_hbm.at[pl.ds(base, _SC_CH)], buf, sem)

    cp0 = _issue(rbase, 0, buf0, sem0)
    for r in range(_SC_RPT):
        row = rbase + r
        cp1 = _issue(row, 1, buf1, sem1)
        ctm_vec = ctm_v[pl.ds(r * 16, 16)]
        cp0.wait()
        acc = _chunk_sum(buf0, ctm_vec, jnp.zeros((16,), jnp.float32))
        if r + 1 < _SC_RPT:
            cp0 = _issue(row + 1, 0, buf0, sem0)
        cp1.wait()
        acc = _chunk_sum(buf1, ctm_vec, acc)
        out_v[pl.ds(r * 16, 16)] = acc
    pltpu.sync_copy(out_v, out_hbm.at[pl.ds((rbase - _TC_ROWS) * 16,
                                            _SC_RPT * 16)])


def _sc_rowsum(ctm, tnew, logits):
    run = pl.kernel(
        _sc_rowsum_body,
        out_type=jax.ShapeDtypeStruct((_SC_ROWS * 16,), jnp.float32),
        mesh=plsc.VectorSubcoreMesh(core_axis_name="c", subcore_axis_name="s"),
        scratch_types=[
            pltpu.VMEM((_SC_RPT * 16,), jnp.float32),
            pltpu.VMEM((16,), jnp.float32),
            pltpu.VMEM((_SC_CH,), jnp.float32),
            pltpu.VMEM((_SC_CH,), jnp.float32),
            pltpu.VMEM((_SC_RPT * 16,), jnp.float32),
            pltpu.SemaphoreType.DMA,
            pltpu.SemaphoreType.DMA,
        ],
    )
    return run(ctm.reshape(_B * 16), tnew.reshape(16), logits.reshape(_B * _C))


def _tc_loss_body(tfull_ref, tblk_ref, t_ref, logits_ref, lacc_out_ref,
                  tnew_ref, lacc_ref):
    i = pl.program_id(0)

    @pl.when(i == 0)
    def _prep():
        tnew_ref[0] = 0.01 * jnp.mean(tfull_ref[:, :]) + 0.99 * t_ref[0]
        lacc_ref[0] = 0.0

    t_new = tnew_ref[0]
    tl = tblk_ref[:, :]
    sin_theta = jnp.sqrt(1.0 - tl * tl)
    ctm = tl * _COS_M - sin_theta * _SIN_M
    final = jnp.where(tl > _THRESHOLD, ctm, tl - _MM)
    num = _S * final
    mod_tl = jnp.where(tl > ctm, tl * (t_new + tl), tl)
    corr = jnp.exp(num) - jnp.exp(_S * mod_tl)

    x = logits_ref[:, :]
    xm = jnp.where(x > ctm, x * (t_new + x), x)
    e = jnp.exp(_S * xm)
    rowsum = jnp.sum(e, axis=1, keepdims=True) + corr
    lacc_ref[0] += jnp.sum(num - jnp.log(rowsum))

    @pl.when(i == _TC_ROWS // _RB - 1)
    def _finish():
        lacc_out_ref[0] = lacc_ref[0]


def _tc_loss_partial(target2d, t, logits):
    return pl.pallas_call(
        _tc_loss_body,
        grid=(_TC_ROWS // _RB,),
        in_specs=[
            pl.BlockSpec((_B, 1), lambda i: (0, 0)),
            pl.BlockSpec((_RB, 1), lambda i: (i, 0)),
            pl.BlockSpec(memory_space=pltpu.SMEM),
            pl.BlockSpec((_RB, _C), lambda i: (i, 0)),
        ],
        out_specs=pl.BlockSpec(memory_space=pltpu.SMEM),
        out_shape=jax.ShapeDtypeStruct((1,), jnp.float32),
        scratch_shapes=[
            pltpu.SMEM((1,), jnp.float32),
            pltpu.SMEM((1,), jnp.float32),
        ],
    )(target2d, target2d, t, logits)


def _tc_epilogue_body(tc_lacc_ref, num_sc_ref, corr_sc_ref, part_sc_ref,
                      loss_ref):
    rowsum = (jnp.sum(part_sc_ref[:, :], axis=1, keepdims=True)
              + corr_sc_ref[:, :])
    l_sc = jnp.sum(num_sc_ref[:, :] - jnp.log(rowsum))
    loss_ref[0] = -(tc_lacc_ref[0] + l_sc) / _B


def _tc_epilogue(tc_lacc, num_sc, corr_sc, part_sc):
    return pl.pallas_call(
        _tc_epilogue_body,
        in_specs=[
            pl.BlockSpec(memory_space=pltpu.SMEM),
            pl.BlockSpec((_SC_ROWS, 1), lambda: (0, 0)),
            pl.BlockSpec((_SC_ROWS, 1), lambda: (0, 0)),
            pl.BlockSpec((_SC_ROWS, 16), lambda: (0, 0)),
        ],
        out_specs=pl.BlockSpec(memory_space=pltpu.SMEM),
        out_shape=jax.ShapeDtypeStruct((1,), jnp.float32),
    )(tc_lacc, num_sc, corr_sc, part_sc)


def kernel(logits, labels, t):
    target = _gather_targets(logits, labels)
    target2d = target.reshape(_B, 1)
    ctm, num, corr, tnew = _tc_prep(target2d, t)
    part_sc = _sc_rowsum(ctm, tnew, logits)
    if _TC_ROWS:
        tc_lacc = _tc_loss_partial(target2d, t, logits)
    else:
        tc_lacc = jnp.zeros((1,), jnp.float32)
    loss = _tc_epilogue(tc_lacc, num[_TC_ROWS:], corr[_TC_ROWS:],
                        part_sc.reshape(_SC_ROWS, 16))
    return loss[0]


# speedup vs baseline: 1.2752x; 1.2752x over previous
import math
import jax
import jax.numpy as jnp
from jax import lax
from jax.experimental import pallas as pl
from jax.experimental.pallas import tpu as pltpu
from jax.experimental.pallas import tpu_sc as plsc

_S = 64.0
_M = 0.5
_COS_M = math.cos(_M)
_SIN_M = math.sin(_M)
_THRESHOLD = math.cos(math.pi - _M)
_MM = math.sin(math.pi - _M) * _M

_B = 1024
_C = 100000

_TC_ROWS = 608
_SC_ROWS = _B - _TC_ROWS
_RB = 32

_NC = 2
_NS = 16
_NW = _NC * _NS
_GPW = _B // _NW
_SC_RPT = _SC_ROWS // _NW if _SC_ROWS else 0
_SC_CH = _C // 2
_SC_VREGS = _SC_CH // 16


def _sc_gather_body(labels_hbm, logits_flat_hbm, out_hbm, lbl_v, idx_v, val_v, sem):
    wid = lax.axis_index("s") * _NC + lax.axis_index("c")
    base = wid * _GPW
    pltpu.sync_copy(labels_hbm.at[pl.ds(base, _GPW)], lbl_v)
    for c in range(_GPW // 16):
        rows = base + c * 16 + lax.iota(jnp.int32, 16)
        idx_v[pl.ds(c * 16, 16)] = rows * _C + lbl_v[pl.ds(c * 16, 16)]
    pltpu.async_copy(logits_flat_hbm.at[idx_v], val_v, sem).wait()
    pltpu.sync_copy(val_v, out_hbm.at[pl.ds(base, _GPW)])


def _gather_targets(logits, labels):
    run = pl.kernel(
        _sc_gather_body,
        out_type=jax.ShapeDtypeStruct((_B,), jnp.float32),
        mesh=plsc.VectorSubcoreMesh(core_axis_name="c", subcore_axis_name="s"),
        scratch_types=[
            pltpu.VMEM((_GPW,), jnp.int32),
            pltpu.VMEM((_GPW,), jnp.int32),
            pltpu.VMEM((_GPW,), jnp.float32),
            pltpu.SemaphoreType.DMA,
        ],
    )
    return run(labels, logits.reshape(_B * _C))


def _tc_prep_body(target_ref, t_ref, ctm_ref, num_ref, corr_ref, tnew_ref):
    tl = target_ref[:, :]
    t_new = 0.01 * jnp.mean(tl) + 0.99 * t_ref[0]
    sin_theta = jnp.sqrt(1.0 - tl * tl)
    ctm = tl * _COS_M - sin_theta * _SIN_M
    final = jnp.where(tl > _THRESHOLD, ctm, tl - _MM)
    num = _S * final
    mod_tl = jnp.where(tl > ctm, tl * (t_new + tl), tl)
    ctm_ref[:, :] = jnp.broadcast_to(ctm, (_B, 16))
    num_ref[:, :] = num
    corr_ref[:, :] = jnp.exp(num) - jnp.exp(_S * mod_tl)
    tnew_ref[:, :] = t_new * jnp.ones((1, 16), jnp.float32)


def _tc_prep(target2d, t):
    return pl.pallas_call(
        _tc_prep_body,
        in_specs=[
            pl.BlockSpec((_B, 1), lambda: (0, 0)),
            pl.BlockSpec(memory_space=pltpu.SMEM),
        ],
        out_specs=[
            pl.BlockSpec((_B, 16), lambda: (0, 0)),
            pl.BlockSpec((_B, 1), lambda: (0, 0)),
            pl.BlockSpec((_B, 1), lambda: (0, 0)),
            pl.BlockSpec((1, 16), lambda: (0, 0)),
        ],
        out_shape=[
            jax.ShapeDtypeStruct((_B, 16), jnp.float32),
            jax.ShapeDtypeStruct((_B, 1), jnp.float32),
            jax.ShapeDtypeStruct((_B, 1), jnp.float32),
            jax.ShapeDtypeStruct((1, 16), jnp.float32),
        ],
    )(target2d, t)


def _sc_rowsum_body(ctm_hbm, tnew_hbm, logits_flat_hbm, out_hbm,
                    ctm_v, tnew_v, buf0, buf1, out_v, sem0, sem1):
    wid = lax.axis_index("s") * _NC + lax.axis_index("c")
    rbase = _TC_ROWS + wid * _SC_RPT
    pltpu.sync_copy(ctm_hbm.at[pl.ds(rbase * 16, _SC_RPT * 16)], ctm_v)
    pltpu.sync_copy(tnew_hbm, tnew_v)
    tn = tnew_v[...]

    def _chunk_sum(buf, ctm_vec, acc0):
        def inner(j, acc):
            v = buf[pl.ds(j * 16, 16)]
            xm = jnp.where(v > ctm_vec, v * (tn + v), v)
            return acc + jnp.exp(_S * xm)
        return lax.fori_loop(0, _SC_VREGS, inner, acc0)

    def _issue(row, half, buf, sem):
        base = row * _C + half * _SC_CH
        return pltpu.async_copy(logits_flat_hbm.at[pl.ds(base, _SC_CH)], buf, sem)

    cp0 = _issue(rbase, 0, buf0, sem0)
    for r in range(_SC_RPT):
        row = rbase + r
        cp1 = _issue(row, 1, buf1, sem1)
        ctm_vec = ctm_v[pl.ds(r * 16, 16)]
        cp0.wait()
        acc = _chunk_sum(buf0, ctm_vec, jnp.zeros((16,), jnp.float32))
        if r + 1 < _SC_RPT:
            cp0 = _issue(row + 1, 0, buf0, sem0)
        cp1.wait()
        acc = _chunk_sum(buf1, ctm_vec, acc)
        out_v[pl.ds(r * 16, 16)] = acc
    pltpu.sync_copy(out_v, out_hbm.at[pl.ds((rbase - _TC_ROWS) * 16,
                                            _SC_RPT * 16)])


def _sc_rowsum(ctm, tnew, logits):
    run = pl.kernel(
        _sc_rowsum_body,
        out_type=jax.ShapeDtypeStruct((_SC_ROWS * 16,), jnp.float32),
        mesh=plsc.VectorSubcoreMesh(core_axis_name="c", subcore_axis_name="s"),
        scratch_types=[
            pltpu.VMEM((_SC_RPT * 16,), jnp.float32),
            pltpu.VMEM((16,), jnp.float32),
            pltpu.VMEM((_SC_CH,), jnp.float32),
            pltpu.VMEM((_SC_CH,), jnp.float32),
            pltpu.VMEM((_SC_RPT * 16,), jnp.float32),
            pltpu.SemaphoreType.DMA,
            pltpu.SemaphoreType.DMA,
        ],
    )
    return run(ctm.reshape(_B * 16), tnew.reshape(16), logits.reshape(_B * _C))


def _tc_loss_body(tfull_ref, tblk_ref, t_ref, logits_ref, lacc_out_ref,
                  tnew_ref, lacc_ref):
    i = pl.program_id(0)

    @pl.when(i == 0)
    def _prep():
        tnew_ref[0] = 0.01 * jnp.mean(tfull_ref[:, :]) + 0.99 * t_ref[0]
        lacc_ref[0] = 0.0

    t_new = tnew_ref[0]
    tl = tblk_ref[:, :]
    sin_theta = jnp.sqrt(1.0 - tl * tl)
    ctm = tl * _COS_M - sin_theta * _SIN_M
    final = jnp.where(tl > _THRESHOLD, ctm, tl - _MM)
    num = _S * final
    mod_tl = jnp.where(tl > ctm, tl * (t_new + tl), tl)
    corr = jnp.exp(num) - jnp.exp(_S * mod_tl)

    x = logits_ref[:, :]
    xm = jnp.where(x > ctm, x * (t_new + x), x)
    e = jnp.exp(_S * xm)
    rowsum = jnp.sum(e, axis=1, keepdims=True) + corr
    lacc_ref[0] += jnp.sum(num - jnp.log(rowsum))

    @pl.when(i == _TC_ROWS // _RB - 1)
    def _finish():
        lacc_out_ref[0] = lacc_ref[0]


def _tc_loss_partial(target2d, t, logits):
    return pl.pallas_call(
        _tc_loss_body,
        grid=(_TC_ROWS // _RB,),
        in_specs=[
            pl.BlockSpec((_B, 1), lambda i: (0, 0)),
            pl.BlockSpec((_RB, 1), lambda i: (i, 0)),
            pl.BlockSpec(memory_space=pltpu.SMEM),
            pl.BlockSpec((_RB, _C), lambda i: (i, 0)),
        ],
        out_specs=pl.BlockSpec(memory_space=pltpu.SMEM),
        out_shape=jax.ShapeDtypeStruct((1,), jnp.float32),
        scratch_shapes=[
            pltpu.SMEM((1,), jnp.float32),
            pltpu.SMEM((1,), jnp.float32),
        ],
    )(target2d, target2d, t, logits)


def _tc_epilogue_body(tc_lacc_ref, num_sc_ref, corr_sc_ref, part_sc_ref,
                      loss_ref):
    rowsum = (jnp.sum(part_sc_ref[:, :], axis=1, keepdims=True)
              + corr_sc_ref[:, :])
    l_sc = jnp.sum(num_sc_ref[:, :] - jnp.log(rowsum))
    loss_ref[0] = -(tc_lacc_ref[0] + l_sc) / _B


def _tc_epilogue(tc_lacc, num_sc, corr_sc, part_sc):
    return pl.pallas_call(
        _tc_epilogue_body,
        in_specs=[
            pl.BlockSpec(memory_space=pltpu.SMEM),
            pl.BlockSpec((_SC_ROWS, 1), lambda: (0, 0)),
            pl.BlockSpec((_SC_ROWS, 1), lambda: (0, 0)),
            pl.BlockSpec((_SC_ROWS, 16), lambda: (0, 0)),
        ],
        out_specs=pl.BlockSpec(memory_space=pltpu.SMEM),
        out_shape=jax.ShapeDtypeStruct((1,), jnp.float32),
    )(tc_lacc, num_sc, corr_sc, part_sc)


def kernel(logits, labels, t):
    target = _gather_targets(logits, labels)
    target2d = target.reshape(_B, 1)
    ctm, num, corr, tnew = _tc_prep(target2d, t)
    part_sc = _sc_rowsum(ctm, tnew, logits)
    if _TC_ROWS:
        tc_lacc = _tc_loss_partial(target2d, t, logits)
    else:
        tc_lacc = jnp.zeros((1,), jnp.float32)
    loss = _tc_epilogue(tc_lacc, num[_TC_ROWS:], corr[_TC_ROWS:],
                        part_sc.reshape(_SC_ROWS, 16))
    return loss[0]


# speedup vs baseline: 2.7722x; 2.1739x over previous
import math
import jax
import jax.numpy as jnp
from jax import lax
from jax.experimental import pallas as pl
from jax.experimental.pallas import tpu as pltpu

_S = 64.0
_M = 0.5
_COS_M = math.cos(_M)
_SIN_M = math.sin(_M)
_THRESHOLD = math.cos(math.pi - _M)
_MM = math.sin(math.pi - _M) * _M

_B = 1024
_C = 100000
_RB = 32
_NRB = _B // _RB
_GR = 8


def _gather_body(lbl_ref, *refs):
    blks, out_ref, acc_ref = refs[:_GR], refs[_GR], refs[_GR + 1]
    i = pl.program_id(0)
    row_iota = lax.broadcasted_iota(jnp.int32, (_GR, 128), 0)
    col_iota = lax.broadcasted_iota(jnp.int32, (_GR, 128), 1)

    @pl.when(i == 0)
    def _init():
        acc_ref[:, :] = jnp.zeros_like(acc_ref)

    vals = []
    for k in range(_GR):
        lbl = lbl_ref[_GR * i + k]
        m = (row_iota == k) & (col_iota == lbl % 128)
        vals.append(jnp.sum(jnp.where(m, blks[k][:, :], 0.0)))
    col = jnp.stack(vals).reshape(_GR, 1)
    acc_ref[:, :] += jnp.where(col_iota == i, col, 0.0)

    @pl.when(i == _B // _GR - 1)
    def _finish():
        out_ref[:, :] = acc_ref[:, :]


def _gather_targets(logits, labels):
    specs = [
        pl.BlockSpec((_GR, 128), lambda i, lbl, k=k: (i, lbl[_GR * i + k] // 128))
        for k in range(_GR)
    ]
    out = pl.pallas_call(
        _gather_body,
        grid_spec=pltpu.PrefetchScalarGridSpec(
            num_scalar_prefetch=1,
            grid=(_B // _GR,),
            in_specs=specs,
            out_specs=pl.BlockSpec((_GR, _B // _GR), lambda i, lbl: (0, 0)),
            scratch_shapes=[pltpu.VMEM((_GR, _B // _GR), jnp.float32)],
        ),
        out_shape=jax.ShapeDtypeStruct((_GR, _B // _GR), jnp.float32),
    )(labels, *([logits] * _GR))
    return out.T.reshape(_B, 1)


def _loss_body(tfull_ref, tblk_ref, t_ref, logits_ref, loss_ref,
               tnew_ref, lacc_ref):
    i = pl.program_id(0)

    @pl.when(i == 0)
    def _prep():
        tnew_ref[0] = 0.01 * jnp.mean(tfull_ref[:, :]) + 0.99 * t_ref[0]
        lacc_ref[0] = 0.0

    t_new = tnew_ref[0]
    tl = tblk_ref[:, :]
    sin_theta = jnp.sqrt(1.0 - tl * tl)
    ctm = tl * _COS_M - sin_theta * _SIN_M
    final = jnp.where(tl > _THRESHOLD, ctm, tl - _MM)
    num = _S * final
    mod_tl = jnp.where(tl > ctm, tl * (t_new + tl), tl)
    corr = jnp.exp(num) - jnp.exp(_S * mod_tl)

    x = logits_ref[:, :]
    xm = jnp.where(x > ctm, x * (t_new + x), x)
    e = jnp.exp(_S * xm)
    rowsum = jnp.sum(e, axis=1, keepdims=True) + corr
    lacc_ref[0] += jnp.sum(num - jnp.log(rowsum))

    @pl.when(i == _NRB - 1)
    def _finish():
        loss_ref[0] = -lacc_ref[0] / _B


def _tc_loss(target2d, t, logits):
    return pl.pallas_call(
        _loss_body,
        grid=(_NRB,),
        in_specs=[
            pl.BlockSpec((_B, 1), lambda i: (0, 0)),
            pl.BlockSpec((_RB, 1), lambda i: (i, 0)),
            pl.BlockSpec(memory_space=pltpu.SMEM),
            pl.BlockSpec((_RB, _C), lambda i: (i, 0)),
        ],
        out_specs=pl.BlockSpec(memory_space=pltpu.SMEM),
        out_shape=jax.ShapeDtypeStruct((1,), jnp.float32),
        scratch_shapes=[
            pltpu.SMEM((1,), jnp.float32),
            pltpu.SMEM((1,), jnp.float32),
        ],
    )(target2d, target2d, t, logits)


def kernel(logits, labels, t):
    target2d = _gather_targets(logits, labels)
    loss = _tc_loss(target2d, t, logits)
    return loss[0]


# speedup vs baseline: 6.4229x; 2.3169x over previous
import math
import jax
import jax.numpy as jnp
from jax import lax
from jax.experimental import pallas as pl
from jax.experimental.pallas import tpu as pltpu

_S = 64.0
_M = 0.5
_COS_M = math.cos(_M)
_SIN_M = math.sin(_M)
_THRESHOLD = math.cos(math.pi - _M)
_MM = math.sin(math.pi - _M) * _M

_B = 1024
_C = 100000
_CBJ = 2048
_NJ = (_C + _CBJ - 1) // _CBJ
_GR = 8


def _gather_body(lbl_ref, *refs):
    blks, out_ref, acc_ref = refs[:_GR], refs[_GR], refs[_GR + 1]
    i = pl.program_id(0)
    io0 = lax.broadcasted_iota(jnp.int32, (_GR, 128), 0)
    io1 = lax.broadcasted_iota(jnp.int32, (_GR, 128), 1)

    @pl.when(i == 0)
    def _init():
        acc_ref[:, :] = jnp.zeros_like(acc_ref)

    upd = jnp.zeros((_GR, 128), jnp.float32)
    for k in range(_GR):
        r = _GR * i + k
        lbl = lbl_ref[r]
        m = (io0 == lbl % _GR) & (io1 == r % 128)
        val = jnp.sum(jnp.where(m, blks[k][:, :], 0.0))
        slot = (io0 == r // 128) & (io1 == r % 128)
        upd = upd + jnp.where(slot, val, 0.0)
    acc_ref[:, :] += upd

    @pl.when(i == _B // _GR - 1)
    def _finish():
        out_ref[:, :] = acc_ref[:, :]


def _gather_targets(logits_t, labels):
    specs = [
        pl.BlockSpec((_GR, 128),
                     lambda i, lbl, k=k: (lbl[_GR * i + k] // _GR, i // 16))
        for k in range(_GR)
    ]
    out = pl.pallas_call(
        _gather_body,
        grid_spec=pltpu.PrefetchScalarGridSpec(
            num_scalar_prefetch=1,
            grid=(_B // _GR,),
            in_specs=specs,
            out_specs=pl.BlockSpec((_GR, 128), lambda i, lbl: (0, 0)),
            scratch_shapes=[pltpu.VMEM((_GR, 128), jnp.float32)],
        ),
        out_shape=jax.ShapeDtypeStruct((_GR, 128), jnp.float32),
    )(labels, *([logits_t] * _GR))
    return out.reshape(1, _B)


def _loss_body(tgt_ref, t_ref, x_ref, loss_ref, tnew_ref, acc_ref):
    j = pl.program_id(0)

    @pl.when(j == 0)
    def _prep():
        tnew_ref[0] = 0.01 * jnp.mean(tgt_ref[:, :]) + 0.99 * t_ref[0]
        acc_ref[:, :] = jnp.zeros_like(acc_ref)

    t_new = tnew_ref[0]
    tl = tgt_ref[:, :]
    sin_theta = jnp.sqrt(1.0 - tl * tl)
    ctm = tl * _COS_M - sin_theta * _SIN_M

    x = x_ref[:, :]
    xm = jnp.where(x > ctm, x * (t_new + x), x)
    e = jnp.exp(_S * xm)

    @pl.when(j < _NJ - 1)
    def _accum():
        acc_ref[:, :] += jnp.sum(e, axis=0, keepdims=True)

    @pl.when(j == _NJ - 1)
    def _finish():
        valid = (j * _CBJ + lax.broadcasted_iota(jnp.int32, e.shape, 0)) < _C
        tail = jnp.sum(jnp.where(valid, e, 0.0), axis=0, keepdims=True)
        final = jnp.where(tl > _THRESHOLD, ctm, tl - _MM)
        num = _S * final
        mod_tl = jnp.where(tl > ctm, tl * (t_new + tl), tl)
        corr = jnp.exp(num) - jnp.exp(_S * mod_tl)
        rowsum = acc_ref[:, :] + tail + corr
        loss_ref[0] = -jnp.mean(num - jnp.log(rowsum))


def _tc_loss(target, t, logits_t):
    return pl.pallas_call(
        _loss_body,
        grid=(_NJ,),
        in_specs=[
            pl.BlockSpec((1, _B), lambda j: (0, 0)),
            pl.BlockSpec(memory_space=pltpu.SMEM),
            pl.BlockSpec((_CBJ, _B), lambda j: (j, 0)),
        ],
        out_specs=pl.BlockSpec(memory_space=pltpu.SMEM),
        out_shape=jax.ShapeDtypeStruct((1,), jnp.float32),
        scratch_shapes=[
            pltpu.SMEM((1,), jnp.float32),
            pltpu.VMEM((1, _B), jnp.float32),
        ],
    )(target, t, logits_t)


def kernel(logits, labels, t):
    logits_t = logits.T
    target = _gather_targets(logits_t, labels)
    loss = _tc_loss(target, t, logits_t)
    return loss[0]


# speedup vs baseline: 7.3717x; 1.1477x over previous
import math
import jax
import jax.numpy as jnp
from jax import lax
from jax.experimental import pallas as pl
from jax.experimental.pallas import tpu as pltpu

_S = 64.0
_M = 0.5
_COS_M = math.cos(_M)
_SIN_M = math.sin(_M)
_THRESHOLD = math.cos(math.pi - _M)
_MM = math.sin(math.pi - _M) * _M

_B = 1024
_C = 100000
_CBJ = 2048
_NJ = (_C + _CBJ - 1) // _CBJ
_GR = 32


def _gather_body(lbl_ref, *refs):
    blks, out_ref, acc_ref = refs[:_GR], refs[_GR], refs[_GR + 1]
    i = pl.program_id(0)
    io0 = lax.broadcasted_iota(jnp.int32, (8, 128), 0)
    io1 = lax.broadcasted_iota(jnp.int32, (8, 128), 1)

    @pl.when(i == 0)
    def _init():
        acc_ref[:, :] = jnp.zeros_like(acc_ref)

    upd = jnp.zeros((8, 128), jnp.float32)
    for k in range(_GR):
        r = _GR * i + k
        lbl = lbl_ref[r]
        m = (io0 == lbl % 8) & (io1 == r % 128)
        val = jnp.sum(jnp.where(m, blks[k][:, :], 0.0))
        slot = (io0 == r // 128) & (io1 == r % 128)
        upd = upd + jnp.where(slot, val, 0.0)
    acc_ref[:, :] += upd

    @pl.when(i == _B // _GR - 1)
    def _finish():
        out_ref[:, :] = acc_ref[:, :]


def _gather_targets(logits_t, labels):
    itile = 128 // _GR
    specs = [
        pl.BlockSpec((8, 128),
                     lambda i, lbl, k=k: (lbl[_GR * i + k] // 8, i // itile))
        for k in range(_GR)
    ]
    out = pl.pallas_call(
        _gather_body,
        grid_spec=pltpu.PrefetchScalarGridSpec(
            num_scalar_prefetch=1,
            grid=(_B // _GR,),
            in_specs=specs,
            out_specs=pl.BlockSpec((8, 128), lambda i, lbl: (0, 0)),
            scratch_shapes=[pltpu.VMEM((8, 128), jnp.float32)],
        ),
        out_shape=jax.ShapeDtypeStruct((8, 128), jnp.float32),
    )(labels, *([logits_t] * _GR))
    return out.reshape(1, _B)


def _loss_body(tgt_ref, t_ref, x_ref, loss_ref, tnew_ref, acc_ref):
    j = pl.program_id(0)

    @pl.when(j == 0)
    def _prep():
        tnew_ref[0] = 0.01 * jnp.mean(tgt_ref[:, :]) + 0.99 * t_ref[0]
        acc_ref[:, :] = jnp.zeros_like(acc_ref)

    t_new = tnew_ref[0]
    tl = tgt_ref[:, :]
    sin_theta = jnp.sqrt(1.0 - tl * tl)
    ctm = tl * _COS_M - sin_theta * _SIN_M

    x = x_ref[:, :]
    xm = jnp.where(x > ctm, x * (t_new + x), x)
    e = jnp.exp(_S * xm)

    @pl.when(j < _NJ - 1)
    def _accum():
        acc_ref[:, :] += jnp.sum(e, axis=0, keepdims=True)

    @pl.when(j == _NJ - 1)
    def _finish():
        valid = (j * _CBJ + lax.broadcasted_iota(jnp.int32, e.shape, 0)) < _C
        tail = jnp.sum(jnp.where(valid, e, 0.0), axis=0, keepdims=True)
        final = jnp.where(tl > _THRESHOLD, ctm, tl - _MM)
        num = _S * final
        mod_tl = jnp.where(tl > ctm, tl * (t_new + tl), tl)
        corr = jnp.exp(num) - jnp.exp(_S * mod_tl)
        rowsum = acc_ref[:, :] + tail + corr
        loss_ref[0] = -jnp.mean(num - jnp.log(rowsum))


def _tc_loss(target, t, logits_t):
    return pl.pallas_call(
        _loss_body,
        grid=(_NJ,),
        in_specs=[
            pl.BlockSpec((1, _B), lambda j: (0, 0)),
            pl.BlockSpec(memory_space=pltpu.SMEM),
            pl.BlockSpec((_CBJ, _B), lambda j: (j, 0)),
        ],
        out_specs=pl.BlockSpec(memory_space=pltpu.SMEM),
        out_shape=jax.ShapeDtypeStruct((1,), jnp.float32),
        scratch_shapes=[
            pltpu.SMEM((1,), jnp.float32),
            pltpu.VMEM((1, _B), jnp.float32),
        ],
    )(target, t, logits_t)


def kernel(logits, labels, t):
    logits_t = logits.T
    target = _gather_targets(logits_t, labels)
    loss = _tc_loss(target, t, logits_t)
    return loss[0]
